# baseline (speedup 1.0000x reference)
import functools

import jax
import jax.numpy as jnp
from jax import lax
from jax.experimental import pallas as pl
from jax.experimental.pallas import tpu as pltpu
from jax.experimental.pallas import tpu_sc as plsc

D_MODEL = 64
SCALE = 8.0
LANES = 16

NC = 2
NS = 16
NW = NC * NS
CHUNK = 128
NBUF = 4


@functools.cache
def _pairify(vocab):
    ntc = vocab // 128
    rem = vocab - ntc * 128
    per_w = (ntc + NW - 1) // NW
    mesh = plsc.VectorSubcoreMesh(core_axis_name="c", subcore_axis_name="s")

    scratch = [pltpu.VMEM((D_MODEL, 128), jnp.float32) for _ in range(2)]
    scratch += [pltpu.VMEM((64, 128), jnp.float32) for _ in range(2)]
    scratch += [pltpu.SemaphoreType.DMA for _ in range(4)]

    @functools.partial(
        pl.kernel,
        mesh=mesh,
        out_type=jax.ShapeDtypeStruct((vocab // 2, 128), jnp.float32),
        scratch_types=scratch,
        compiler_params=pltpu.CompilerParams(needs_layout_passes=False),
    )
    def pairify(tt_hbm, out_hbm, *rest):
        ibuf = rest[:2]
        obuf = rest[2:4]
        isem = rest[4:6]
        osem = rest[6:8]

        wid = lax.axis_index("s") * NC + lax.axis_index("c")
        iota = lax.iota(jnp.int32, LANES)

        def col_of(g):
            c = wid + NW * g
            return lax.select(c < ntc, c, 0)

        def start_in(g, b):
            c = col_of(g)
            pltpu.async_copy(
                tt_hbm.at[:, pl.ds(c * 128, 128)], ibuf[b], isem[b])

        for b in range(2):
            start_in(b, b)

        def step2(grp, carry):
          for b in range(2):
            g = grp * 2 + b
            c = col_of(g)
            pltpu.make_async_copy(
                tt_hbm.at[:, pl.ds(c * 128, 128)], ibuf[b], isem[b]).wait()

            @pl.when(grp > 0)
            def _wait_out():
                pltpu.make_async_copy(
                    obuf[b], out_hbm.at[pl.ds(0, 64)], osem[b]).wait()

            @plsc.parallel_loop(0, 64, unroll=8)
            def rowq(q):
                even = iota * 0 + 2 * q
                for dh in range(D_MODEL // LANES):
                    rows = dh * LANES + iota
                    obuf[b][q, pl.ds(dh * LANES, LANES)] = (
                        plsc.load_gather(ibuf[b], [rows, even]) * SCALE)
                    obuf[b][q, pl.ds(D_MODEL + dh * LANES, LANES)] = (
                        plsc.load_gather(ibuf[b], [rows, even + 1]) * SCALE)

            start_in(g + 2, b)

            pltpu.async_copy(
                obuf[b], out_hbm.at[pl.ds(c * 64, 64)], osem[b])
          return carry

        ngrp2 = (per_w + 1) // 2
        lax.fori_loop(0, ngrp2, step2, 0)
        for b in range(2):
            c = col_of(ngrp2 * 2 + b)
            pltpu.make_async_copy(
                tt_hbm.at[:, pl.ds(c * 128, 128)], ibuf[b], isem[b]).wait()
            pltpu.make_async_copy(
                obuf[b], out_hbm.at[pl.ds(0, 64)], osem[b]).wait()

    return pairify


@functools.cache
def _build(batch, npos):
    mesh = plsc.VectorSubcoreMesh(core_axis_name="c", subcore_axis_name="s")

    scratch = [pltpu.VMEM((npos, CHUNK), jnp.int32)]
    scratch += [pltpu.VMEM((CHUNK,), jnp.int32) for _ in range(NBUF)]
    scratch += [pltpu.VMEM((CHUNK, 128), jnp.float32) for _ in range(NBUF)]
    scratch += [pltpu.VMEM((D_MODEL, CHUNK), jnp.float32) for _ in range(NBUF)]
    scratch += [pltpu.SemaphoreType.DMA for _ in range(2 * NBUF)]

    @functools.partial(
        pl.kernel,
        mesh=mesh,
        out_type=jax.ShapeDtypeStruct((npos, D_MODEL, batch), jnp.float32),
        scratch_types=scratch,
        compiler_params=pltpu.CompilerParams(needs_layout_passes=False),
    )
    def emb(xt_hbm, table_hbm, out_hbm, idx_v, *rest):
        gidx = rest[:NBUF]
        gbuf = rest[NBUF:2 * NBUF]
        tbuf = rest[2 * NBUF:3 * NBUF]
        gsem = rest[3 * NBUF:4 * NBUF]
        osem = rest[4 * NBUF:5 * NBUF]

        wid = lax.axis_index("s") * NC + lax.axis_index("c")
        iota = lax.iota(jnp.int32, LANES)

        pltpu.sync_copy(xt_hbm.at[:, pl.ds(wid * CHUNK, CHUNK)], idx_v)

        def start_gather(j, b):
            def halve(v, carry):
                sl = pl.ds(LANES * v, LANES)
                gidx[b][sl] = lax.shift_right_logical(idx_v[j, sl], 1)
                return carry
            lax.fori_loop(0, CHUNK // LANES, halve, 0, unroll=True)
            pltpu.async_copy(table_hbm.at[gidx[b]], gbuf[b], gsem[b])

        for b in range(NBUF):
            start_gather(b, b)

        def group(g, carry):
            for b in range(NBUF):
                j = g * NBUF + b

                pltpu.make_async_copy(
                    table_hbm.at[gidx[b]], gbuf[b], gsem[b]).wait()

                @pl.when(g > 0)
                def _wait_out():
                    pltpu.make_async_copy(
                        tbuf[b],
                        out_hbm.at[0, :, pl.ds(wid * CHUNK, CHUNK)],
                        osem[b]).wait()

                def grp16(gi, acc):
                    rows = gi * LANES + iota
                    cols = (idx_v[j, pl.ds(gi * LANES, LANES)] & 1) * D_MODEL

                    @plsc.parallel_loop(0, D_MODEL, unroll=16)
                    def feat(d):
                        v = plsc.load_gather(gbuf[b], [rows, cols + d])
                        tbuf[b][d, pl.ds(gi * LANES, LANES)] = v * SCALE
                    return acc
                lax.fori_loop(0, CHUNK // LANES, grp16, 0)

                @pl.when(g < (npos // NBUF) - 1)
                def _next_gather():
                    start_gather(j + NBUF, b)

                pltpu.async_copy(
                    tbuf[b],
                    out_hbm.at[j, :, pl.ds(wid * CHUNK, CHUNK)],
                    osem[b])
            return carry

        lax.fori_loop(0, npos // NBUF, group, 0)

        for b in range(NBUF):
            pltpu.make_async_copy(
                tbuf[b],
                out_hbm.at[0, :, pl.ds(wid * CHUNK, CHUNK)],
                osem[b]).wait()

    return emb


@jax.jit
def kernel(x, table):
    batch, npos = x.shape
    xt = x.T.astype(jnp.int32)
    table2 = jnp.concatenate([table[0::2], table[1::2]], axis=1)
    out = _build(batch, npos)(xt, table2)
    return out.transpose(2, 0, 1)


# speedup vs baseline: 7.3743x; 7.3743x over previous
import functools

import jax
import jax.numpy as jnp
from jax import lax
from jax.experimental import pallas as pl
from jax.experimental.pallas import tpu as pltpu
from jax.experimental.pallas import tpu_sc as plsc

D_MODEL = 64
SCALE = 8.0
LANES = 16

NC = 2
NS = 16
NW = NC * NS
CHUNK = 128
NBUF = 4


@functools.cache
def _pairify(vocab):
    ntc = vocab // 128
    rem = vocab - ntc * 128
    per_w = (ntc + NW - 1) // NW
    mesh = plsc.VectorSubcoreMesh(core_axis_name="c", subcore_axis_name="s")

    scratch = [pltpu.VMEM((D_MODEL, 128), jnp.float32) for _ in range(2)]
    scratch += [pltpu.VMEM((64, 128), jnp.float32) for _ in range(2)]
    scratch += [pltpu.SemaphoreType.DMA for _ in range(4)]

    @functools.partial(
        pl.kernel,
        mesh=mesh,
        out_type=jax.ShapeDtypeStruct((vocab // 2, 128), jnp.float32),
        scratch_types=scratch,
        compiler_params=pltpu.CompilerParams(needs_layout_passes=False),
    )
    def pairify(tt_hbm, out_hbm, *rest):
        ibuf = rest[:2]
        obuf = rest[2:4]
        isem = rest[4:6]
        osem = rest[6:8]

        wid = lax.axis_index("s") * NC + lax.axis_index("c")
        iota = lax.iota(jnp.int32, LANES)

        def col_of(g):
            c = wid + NW * g
            return lax.select(c < ntc, c, 0)

        def start_in(g, b):
            c = col_of(g)
            pltpu.async_copy(
                tt_hbm.at[:, pl.ds(c * 128, 128)], ibuf[b], isem[b])

        for b in range(2):
            start_in(b, b)

        def step2(grp, carry):
          for b in range(2):
            g = grp * 2 + b
            c = col_of(g)
            pltpu.make_async_copy(
                tt_hbm.at[:, pl.ds(c * 128, 128)], ibuf[b], isem[b]).wait()

            @pl.when(grp > 0)
            def _wait_out():
                pltpu.make_async_copy(
                    obuf[b], out_hbm.at[pl.ds(0, 64)], osem[b]).wait()

            @plsc.parallel_loop(0, 64, unroll=8)
            def rowq(q):
                even = iota * 0 + 2 * q
                for dh in range(D_MODEL // LANES):
                    rows = dh * LANES + iota
                    obuf[b][q, pl.ds(dh * LANES, LANES)] = (
                        plsc.load_gather(ibuf[b], [rows, even]) * SCALE)
                    obuf[b][q, pl.ds(D_MODEL + dh * LANES, LANES)] = (
                        plsc.load_gather(ibuf[b], [rows, even + 1]) * SCALE)

            start_in(g + 2, b)

            pltpu.async_copy(
                obuf[b], out_hbm.at[pl.ds(c * 64, 64)], osem[b])
          return carry

        ngrp2 = (per_w + 1) // 2
        lax.fori_loop(0, ngrp2, step2, 0)
        for b in range(2):
            c = col_of(ngrp2 * 2 + b)
            pltpu.make_async_copy(
                tt_hbm.at[:, pl.ds(c * 128, 128)], ibuf[b], isem[b]).wait()
            pltpu.make_async_copy(
                obuf[b], out_hbm.at[pl.ds(0, 64)], osem[b]).wait()

    return pairify


@functools.cache
def _build(batch, npos):
    mesh = plsc.VectorSubcoreMesh(core_axis_name="c", subcore_axis_name="s")

    scratch = [pltpu.VMEM((npos, CHUNK), jnp.int32)]
    scratch += [pltpu.VMEM((CHUNK,), jnp.int32) for _ in range(NBUF)]
    scratch += [pltpu.VMEM((CHUNK, 128), jnp.float32) for _ in range(NBUF)]
    scratch += [pltpu.VMEM((D_MODEL, CHUNK), jnp.float32) for _ in range(NBUF)]
    scratch += [pltpu.SemaphoreType.DMA for _ in range(2 * NBUF)]

    @functools.partial(
        pl.kernel,
        mesh=mesh,
        out_type=jax.ShapeDtypeStruct((npos, D_MODEL, batch), jnp.float32),
        scratch_types=scratch,
        compiler_params=pltpu.CompilerParams(needs_layout_passes=False),
    )
    def emb(xt_hbm, table_hbm, out_hbm, idx_v, *rest):
        gidx = rest[:NBUF]
        gbuf = rest[NBUF:2 * NBUF]
        tbuf = rest[2 * NBUF:3 * NBUF]
        gsem = rest[3 * NBUF:4 * NBUF]
        osem = rest[4 * NBUF:5 * NBUF]

        wid = lax.axis_index("s") * NC + lax.axis_index("c")
        iota = lax.iota(jnp.int32, LANES)

        pltpu.sync_copy(xt_hbm.at[:, pl.ds(wid * CHUNK, CHUNK)], idx_v)

        def start_gather(j, b):
            def halve(v, carry):
                sl = pl.ds(LANES * v, LANES)
                gidx[b][sl] = lax.shift_right_logical(idx_v[j, sl], 1)
                return carry
            lax.fori_loop(0, CHUNK // LANES, halve, 0, unroll=True)
            pltpu.async_copy(table_hbm.at[gidx[b]], gbuf[b], gsem[b])

        for b in range(NBUF):
            start_gather(b, b)

        def group(g, carry):
            for b in range(NBUF):
                j = g * NBUF + b

                pltpu.make_async_copy(
                    table_hbm.at[gidx[b]], gbuf[b], gsem[b]).wait()

                @pl.when(g > 0)
                def _wait_out():
                    pltpu.make_async_copy(
                        tbuf[b],
                        out_hbm.at[0, :, pl.ds(wid * CHUNK, CHUNK)],
                        osem[b]).wait()

                def grp16(gi, acc):
                    rows = gi * LANES + iota
                    cols = (idx_v[j, pl.ds(gi * LANES, LANES)] & 1) * D_MODEL

                    @plsc.parallel_loop(0, D_MODEL, unroll=16)
                    def feat(d):
                        v = plsc.load_gather(gbuf[b], [rows, cols + d])
                        tbuf[b][d, pl.ds(gi * LANES, LANES)] = v * SCALE
                    return acc
                lax.fori_loop(0, CHUNK // LANES, grp16, 0)

                @pl.when(g < (npos // NBUF) - 1)
                def _next_gather():
                    start_gather(j + NBUF, b)

                pltpu.async_copy(
                    tbuf[b],
                    out_hbm.at[j, :, pl.ds(wid * CHUNK, CHUNK)],
                    osem[b])
            return carry

        lax.fori_loop(0, npos // NBUF, group, 0)

        for b in range(NBUF):
            pltpu.make_async_copy(
                tbuf[b],
                out_hbm.at[0, :, pl.ds(wid * CHUNK, CHUNK)],
                osem[b]).wait()

    return emb


@jax.jit
def kernel(x, table):
    batch, npos = x.shape
    xt = x.T.astype(jnp.int32)
    table2 = table.reshape(table.shape[0] // 2, 2 * table.shape[1])
    out = _build(batch, npos)(xt, table2)
    return out.transpose(2, 0, 1)


# speedup vs baseline: 8.0214x; 1.0877x over previous
import functools

import jax
import jax.numpy as jnp
from jax import lax
from jax.experimental import pallas as pl
from jax.experimental.pallas import tpu as pltpu
from jax.experimental.pallas import tpu_sc as plsc

D_MODEL = 64
SCALE = 8.0
LANES = 16

NC = 2
NS = 16
NW = NC * NS
CHUNK = 128
NBUF = 4


@functools.cache
def _pairify(vocab):
    ntc = vocab // 128
    rem = vocab - ntc * 128
    per_w = (ntc + NW - 1) // NW
    mesh = plsc.VectorSubcoreMesh(core_axis_name="c", subcore_axis_name="s")

    scratch = [pltpu.VMEM((D_MODEL, 128), jnp.float32) for _ in range(2)]
    scratch += [pltpu.VMEM((64, 128), jnp.float32) for _ in range(2)]
    scratch += [pltpu.SemaphoreType.DMA for _ in range(4)]

    @functools.partial(
        pl.kernel,
        mesh=mesh,
        out_type=jax.ShapeDtypeStruct((vocab // 2, 128), jnp.float32),
        scratch_types=scratch,
        compiler_params=pltpu.CompilerParams(needs_layout_passes=False),
    )
    def pairify(tt_hbm, out_hbm, *rest):
        ibuf = rest[:2]
        obuf = rest[2:4]
        isem = rest[4:6]
        osem = rest[6:8]

        wid = lax.axis_index("s") * NC + lax.axis_index("c")
        iota = lax.iota(jnp.int32, LANES)

        def col_of(g):
            c = wid + NW * g
            return lax.select(c < ntc, c, 0)

        def start_in(g, b):
            c = col_of(g)
            pltpu.async_copy(
                tt_hbm.at[:, pl.ds(c * 128, 128)], ibuf[b], isem[b])

        for b in range(2):
            start_in(b, b)

        def step2(grp, carry):
          for b in range(2):
            g = grp * 2 + b
            c = col_of(g)
            pltpu.make_async_copy(
                tt_hbm.at[:, pl.ds(c * 128, 128)], ibuf[b], isem[b]).wait()

            @pl.when(grp > 0)
            def _wait_out():
                pltpu.make_async_copy(
                    obuf[b], out_hbm.at[pl.ds(0, 64)], osem[b]).wait()

            @plsc.parallel_loop(0, 64, unroll=8)
            def rowq(q):
                even = iota * 0 + 2 * q
                for dh in range(D_MODEL // LANES):
                    rows = dh * LANES + iota
                    obuf[b][q, pl.ds(dh * LANES, LANES)] = (
                        plsc.load_gather(ibuf[b], [rows, even]) * SCALE)
                    obuf[b][q, pl.ds(D_MODEL + dh * LANES, LANES)] = (
                        plsc.load_gather(ibuf[b], [rows, even + 1]) * SCALE)

            start_in(g + 2, b)

            pltpu.async_copy(
                obuf[b], out_hbm.at[pl.ds(c * 64, 64)], osem[b])
          return carry

        ngrp2 = (per_w + 1) // 2
        lax.fori_loop(0, ngrp2, step2, 0)
        for b in range(2):
            c = col_of(ngrp2 * 2 + b)
            pltpu.make_async_copy(
                tt_hbm.at[:, pl.ds(c * 128, 128)], ibuf[b], isem[b]).wait()
            pltpu.make_async_copy(
                obuf[b], out_hbm.at[pl.ds(0, 64)], osem[b]).wait()

    return pairify


@functools.cache
def _build(batch, npos):
    mesh = plsc.VectorSubcoreMesh(core_axis_name="c", subcore_axis_name="s")

    scratch = [pltpu.VMEM((npos, CHUNK), jnp.int32)]
    scratch += [pltpu.VMEM((CHUNK,), jnp.int32) for _ in range(NBUF)]
    scratch += [pltpu.VMEM((CHUNK, 128), jnp.float32) for _ in range(NBUF)]
    scratch += [pltpu.VMEM((D_MODEL, CHUNK), jnp.float32) for _ in range(NBUF)]
    scratch += [pltpu.SemaphoreType.DMA for _ in range(2 * NBUF)]

    @functools.partial(
        pl.kernel,
        mesh=mesh,
        out_type=jax.ShapeDtypeStruct((npos, D_MODEL, batch), jnp.float32),
        scratch_types=scratch,
        compiler_params=pltpu.CompilerParams(needs_layout_passes=False),
    )
    def emb(xt_hbm, table_hbm, out_hbm, idx_v, *rest):
        gidx = rest[:NBUF]
        gbuf = rest[NBUF:2 * NBUF]
        tbuf = rest[2 * NBUF:3 * NBUF]
        gsem = rest[3 * NBUF:4 * NBUF]
        osem = rest[4 * NBUF:5 * NBUF]

        wid = lax.axis_index("s") * NC + lax.axis_index("c")
        iota = lax.iota(jnp.int32, LANES)

        pltpu.sync_copy(xt_hbm.at[:, pl.ds(wid * CHUNK, CHUNK)], idx_v)

        def start_gather(j, b):
            def halve(v, carry):
                sl = pl.ds(LANES * v, LANES)
                gidx[b][sl] = lax.shift_right_logical(idx_v[j, sl], 1)
                return carry
            lax.fori_loop(0, CHUNK // LANES, halve, 0, unroll=True)
            pltpu.async_copy(table_hbm.at[gidx[b]], gbuf[b], gsem[b])

        for b in range(NBUF):
            start_gather(b, b)

        def group(g, carry):
            for b in range(NBUF):
                j = g * NBUF + b

                pltpu.make_async_copy(
                    table_hbm.at[gidx[b]], gbuf[b], gsem[b]).wait()

                @pl.when(g > 0)
                def _wait_out():
                    pltpu.make_async_copy(
                        tbuf[b],
                        out_hbm.at[0, :, pl.ds(wid * CHUNK, CHUNK)],
                        osem[b]).wait()

                def grp16(gi, acc):
                    rows = gi * LANES + iota
                    par = (idx_v[j, pl.ds(gi * LANES, LANES)] & 1) * D_MODEL
                    for dh in range(D_MODEL // LANES):
                        for k in range(LANES):
                            drow = dh * LANES + ((iota + k) & (LANES - 1))
                            v = plsc.load_gather(gbuf[b], [rows, par + drow])
                            plsc.store_scatter(
                                tbuf[b], [drow, rows], v * SCALE)
                    return acc
                lax.fori_loop(0, CHUNK // LANES, grp16, 0)

                @pl.when(g < (npos // NBUF) - 1)
                def _next_gather():
                    start_gather(j + NBUF, b)

                pltpu.async_copy(
                    tbuf[b],
                    out_hbm.at[j, :, pl.ds(wid * CHUNK, CHUNK)],
                    osem[b])
            return carry

        lax.fori_loop(0, npos // NBUF, group, 0)

        for b in range(NBUF):
            pltpu.make_async_copy(
                tbuf[b],
                out_hbm.at[0, :, pl.ds(wid * CHUNK, CHUNK)],
                osem[b]).wait()

    return emb


@jax.jit
def kernel(x, table):
    batch, npos = x.shape
    xt = x.T.astype(jnp.int32)
    table2 = table.reshape(table.shape[0] // 2, 2 * table.shape[1])
    out = _build(batch, npos)(xt, table2)
    return out.transpose(2, 0, 1)


# speedup vs baseline: 8.0246x; 1.0004x over previous
import functools

import jax
import jax.numpy as jnp
from jax import lax
from jax.experimental import pallas as pl
from jax.experimental.pallas import tpu as pltpu
from jax.experimental.pallas import tpu_sc as plsc

D_MODEL = 64
SCALE = 8.0
LANES = 16

NC = 2
NS = 16
NW = NC * NS
CHUNK = 128
NBUF = 4


@functools.cache
def _build(batch, npos):
    mesh = plsc.VectorSubcoreMesh(core_axis_name="c", subcore_axis_name="s")

    scratch = [pltpu.VMEM((npos, CHUNK), jnp.int32)]
    scratch += [pltpu.VMEM((CHUNK,), jnp.int32) for _ in range(NBUF)]
    scratch += [pltpu.VMEM((CHUNK, 128), jnp.float32) for _ in range(NBUF)]
    scratch += [pltpu.VMEM((D_MODEL, CHUNK), jnp.float32) for _ in range(NBUF)]
    scratch += [pltpu.SemaphoreType.DMA for _ in range(2 * NBUF)]

    @functools.partial(
        pl.kernel,
        mesh=mesh,
        out_type=jax.ShapeDtypeStruct((npos, D_MODEL, batch), jnp.float32),
        scratch_types=scratch,
        compiler_params=pltpu.CompilerParams(needs_layout_passes=False),
    )
    def emb(xt_hbm, table_hbm, out_hbm, idx_v, *rest):
        gidx = rest[:NBUF]
        gbuf = rest[NBUF:2 * NBUF]
        tbuf = rest[2 * NBUF:3 * NBUF]
        gsem = rest[3 * NBUF:4 * NBUF]
        osem = rest[4 * NBUF:5 * NBUF]

        wid = lax.axis_index("s") * NC + lax.axis_index("c")
        iota = lax.iota(jnp.int32, LANES)

        pltpu.sync_copy(xt_hbm.at[:, pl.ds(wid * CHUNK, CHUNK)], idx_v)

        def start_gather(j, b):
            def halve(v, carry):
                sl = pl.ds(LANES * v, LANES)
                gidx[b][sl] = lax.shift_right_logical(idx_v[j, sl], 1)
                return carry
            lax.fori_loop(0, CHUNK // LANES, halve, 0, unroll=True)
            pltpu.async_copy(table_hbm.at[gidx[b]], gbuf[b], gsem[b])

        for b in range(NBUF):
            start_gather(b, b)

        def group(g, carry):
            for b in range(NBUF):
                j = g * NBUF + b

                pltpu.make_async_copy(
                    table_hbm.at[gidx[b]], gbuf[b], gsem[b]).wait()

                @pl.when(g > 0)
                def _wait_out():
                    pltpu.make_async_copy(
                        tbuf[b],
                        out_hbm.at[0, :, pl.ds(wid * CHUNK, CHUNK)],
                        osem[b]).wait()

                def grp16(gi, acc):
                    rows = gi * LANES + iota
                    par = (idx_v[j, pl.ds(gi * LANES, LANES)] & 1) * D_MODEL
                    for dh in range(D_MODEL // LANES):
                        for k in range(LANES):
                            drow = dh * LANES + ((iota + k) & (LANES - 1))
                            v = plsc.load_gather(gbuf[b], [rows, par + drow])
                            plsc.store_scatter(
                                tbuf[b], [drow, rows], v * SCALE)
                    return acc
                lax.fori_loop(0, CHUNK // LANES, grp16, 0)

                @pl.when(g < (npos // NBUF) - 1)
                def _next_gather():
                    start_gather(j + NBUF, b)

                pltpu.async_copy(
                    tbuf[b],
                    out_hbm.at[j, :, pl.ds(wid * CHUNK, CHUNK)],
                    osem[b])
            return carry

        lax.fori_loop(0, npos // NBUF, group, 0)

        for b in range(NBUF):
            pltpu.make_async_copy(
                tbuf[b],
                out_hbm.at[0, :, pl.ds(wid * CHUNK, CHUNK)],
                osem[b]).wait()

    return emb


@jax.jit
def kernel(x, table):
    batch, npos = x.shape
    xt = x.T.astype(jnp.int32)
    table2 = table.reshape(table.shape[0] // 2, 2 * table.shape[1])
    out = _build(batch, npos)(xt, table2)
    return out.transpose(2, 0, 1)
